# baseline (speedup 1.0000x reference)
import math

import jax
import jax.numpy as jnp
from jax import lax
from jax.experimental import pallas as pl
from jax.experimental.pallas import tpu as pltpu

_SIZE = 1000
_PAD = 0
_SMOOTH = 0.1
_CONF = 1.0 - _SMOOTH
_EPS = _SMOOTH / (_SIZE - 2)
_ROW_CONST = _SMOOTH * math.log(_EPS) + _CONF * math.log(_CONF)

_BC = 4096


def _tc_body(xt_ref, t_ref, a_ref, n_ref):
    pid = pl.program_id(0)

    @pl.when(pid == 0)
    def _():
        a_ref[0, 0] = 0.0
        n_ref[0, 0] = 0.0

    xb = xt_ref[...]
    t = t_ref[...]
    mask = t != _PAD
    rows = lax.broadcasted_iota(jnp.int32, xb.shape, 0)
    w = jnp.where(rows == t, -_CONF, -_EPS)
    w = jnp.where((rows == _PAD) | (~mask), 0.0, w)
    a_ref[0, 0] += jnp.sum(xb * w)
    n_ref[0, 0] += jnp.sum(jnp.where(mask, 1.0, 0.0))


def _tc_weighted_sum(xt, t2d):
    n_cols = xt.shape[1]
    scalar_spec = pl.BlockSpec((1, 1), lambda i: (0, 0),
                               memory_space=pltpu.SMEM)
    return pl.pallas_call(
        _tc_body,
        grid=(n_cols // _BC,),
        in_specs=[
            pl.BlockSpec((_SIZE, _BC), lambda i: (0, i)),
            pl.BlockSpec((1, _BC), lambda i: (0, i)),
        ],
        out_specs=[scalar_spec, scalar_spec],
        out_shape=[jax.ShapeDtypeStruct((1, 1), jnp.float32)] * 2,
    )(xt, t2d)


def kernel(x, target):
    n_rows = x.shape[0]
    t32 = target.astype(jnp.int32)
    a, n = _tc_weighted_sum(x.T, t32.reshape(1, n_rows))
    total = n[0, 0] * _ROW_CONST + a[0, 0]
    return total.astype(jnp.float32)
